# Initial kernel scaffold; baseline (speedup 1.0000x reference)
import jax
import jax.numpy as jnp
from jax.experimental import pallas as pl


def kernel(x, edge_index, W1l, W1r, b1, W2l, W2r, b2, Wres, bres, Wmul, Wmur, bmu, Wlsl, Wlsr, bls):
    raise NotImplementedError("write your pallas kernel here")


# speedup vs baseline: 3.3371x; 3.3371x over previous
import jax
import jax.numpy as jnp
from jax import lax
from jax.experimental import pallas as pl
from jax.experimental.pallas import tpu as pltpu
from jax.experimental.pallas import tpu_sc as plsc

N = 10000
E = 320000
D = 128
H = 128
O = 64

NC = 2
NS = 16
NW = NC * NS
K = 128
EPW = 10112
CHUNKS = EPW // K
E_PAD = NW * EPW
N_PAD = 10240
DUMMY = N + 64
RPS = N_PAD // NS
ZB = 16

_mesh = plsc.VectorSubcoreMesh(core_axis_name="c", subcore_axis_name="s")


def _seg_body(h_hbm, src_hbm, dst_hbm, out_hbm, idx_s, idx_d, rows, zf,
              acc, gsem):
    cid = lax.axis_index("c")
    sid = lax.axis_index("s")
    wid = sid * NC + cid

    zero16 = jnp.zeros((16,), dtype=jnp.float32)
    for i in range(ZB):
        for j in range(8):
            zf[i, pl.ds(j * 16, 16)] = zero16

    rbase = sid * RPS
    def zloop(i, _):
        pltpu.sync_copy(zf, acc.at[pl.ds(rbase + i * ZB, ZB)])
        return 0
    lax.fori_loop(0, RPS // ZB, zloop, 0)

    plsc.subcore_barrier()

    ebase = wid * EPW
    def body(i, _):
        off = ebase + i * K
        pltpu.sync_copy(src_hbm.at[pl.ds(off, K)], idx_s)
        pltpu.sync_copy(dst_hbm.at[pl.ds(off, K)], idx_d)
        pltpu.async_copy(h_hbm.at[idx_s], rows, gsem).wait()
        pltpu.sync_copy(rows, acc.at[idx_d], add=True)
        return 0
    lax.fori_loop(0, CHUNKS, body, 0)

    plsc.subcore_barrier()

    pltpu.sync_copy(acc.at[pl.ds(rbase, RPS)],
                    out_hbm.at[pl.ds(cid * N_PAD + rbase, RPS)])


_seg = pl.kernel(
    _seg_body,
    out_type=jax.ShapeDtypeStruct((NC * N_PAD, 128), jnp.float32),
    mesh=_mesh,
    scratch_types=[
        pltpu.VMEM((K,), jnp.int32),
        pltpu.VMEM((K,), jnp.int32),
        pltpu.VMEM((K, 128), jnp.float32),
        pltpu.VMEM((ZB, 128), jnp.float32),
        pltpu.VMEM_SHARED((N_PAD, 128), jnp.float32),
        pltpu.SemaphoreType.DMA,
    ],
)


BN = 1000
GRID = N // BN


def _mean(a_ref, deg_ref):
    s = a_ref[0] + a_ref[1]
    deg = deg_ref[0, :, 0:1] + deg_ref[1, :, 0:1]
    return s * (1.0 / jnp.maximum(deg, 1.0))


def _tc1_body(a_ref, deg_ref, x_ref, wl_ref, wr_ref, b_ref, o_ref):
    mean = _mean(a_ref, deg_ref)
    h = (jnp.dot(mean, wl_ref[...], preferred_element_type=jnp.float32)
         + jnp.dot(x_ref[...], wr_ref[...], preferred_element_type=jnp.float32)
         + b_ref[...])
    o_ref[...] = jnp.maximum(h, 0.0)


def _tc2_body(a_ref, deg_ref, h1_ref, x_ref, wl_ref, wr_ref, b_ref,
              wres_ref, bres_ref, o_ref):
    mean = _mean(a_ref, deg_ref)
    h = (jnp.dot(mean, wl_ref[...], preferred_element_type=jnp.float32)
         + jnp.dot(h1_ref[...], wr_ref[...], preferred_element_type=jnp.float32)
         + b_ref[...])
    o_ref[...] = (jnp.maximum(h, 0.0)
                  + jnp.dot(x_ref[...], wres_ref[...],
                            preferred_element_type=jnp.float32)
                  + bres_ref[...])


def _tc3_body(a_ref, deg_ref, h_ref, wmul_ref, wmur_ref, bmu_ref,
              wlsl_ref, wlsr_ref, bls_ref, mu_ref, ls_ref):
    mean = _mean(a_ref, deg_ref)
    mu_ref[...] = (jnp.dot(mean, wmul_ref[...], preferred_element_type=jnp.float32)
                   + jnp.dot(h_ref[...], wmur_ref[...],
                             preferred_element_type=jnp.float32)
                   + bmu_ref[...])
    ls_ref[...] = (jnp.dot(mean, wlsl_ref[...], preferred_element_type=jnp.float32)
                   + jnp.dot(h_ref[...], wlsr_ref[...],
                             preferred_element_type=jnp.float32)
                   + bls_ref[...])


def _a_spec():
    return pl.BlockSpec((NC, BN, 128), lambda i: (0, i, 0))


def _deg_spec():
    return pl.BlockSpec((NC, BN, 128), lambda i: (0, i, 0))


def _row_spec(width):
    return pl.BlockSpec((BN, width), lambda i: (i, 0))


def _w_spec(r, c):
    return pl.BlockSpec((r, c), lambda i: (0, 0))


def _tc1(a, deg, x, wl, wr, b):
    return pl.pallas_call(
        _tc1_body,
        out_shape=jax.ShapeDtypeStruct((N, H), jnp.float32),
        grid=(GRID,),
        in_specs=[_a_spec(), _deg_spec(), _row_spec(D),
                  _w_spec(D, H), _w_spec(D, H), _w_spec(1, H)],
        out_specs=_row_spec(H),
    )(a, deg, x, wl, wr, b)


def _tc2(a, deg, h1, x, wl, wr, b, wres, bres):
    return pl.pallas_call(
        _tc2_body,
        out_shape=jax.ShapeDtypeStruct((N, H), jnp.float32),
        grid=(GRID,),
        in_specs=[_a_spec(), _deg_spec(), _row_spec(H), _row_spec(D),
                  _w_spec(H, H), _w_spec(H, H), _w_spec(1, H),
                  _w_spec(D, H), _w_spec(1, H)],
        out_specs=_row_spec(H),
    )(a, deg, h1, x, wl, wr, b, wres, bres)


def _tc3(a, deg, h, wmul, wmur, bmu, wlsl, wlsr, bls):
    return pl.pallas_call(
        _tc3_body,
        out_shape=(jax.ShapeDtypeStruct((N, O), jnp.float32),
                   jax.ShapeDtypeStruct((N, O), jnp.float32)),
        grid=(GRID,),
        in_specs=[_a_spec(), _deg_spec(), _row_spec(H),
                  _w_spec(H, O), _w_spec(H, O), _w_spec(1, O),
                  _w_spec(H, O), _w_spec(H, O), _w_spec(1, O)],
        out_specs=(_row_spec(O), _row_spec(O)),
    )(a, deg, h, wmul, wmur, bmu, wlsl, wlsr, bls)


def kernel(x, edge_index, W1l, W1r, b1, W2l, W2r, b2, Wres, bres,
           Wmul, Wmur, bmu, Wlsl, Wlsr, bls):
    src = edge_index[0].astype(jnp.int32)
    dst = edge_index[1].astype(jnp.int32)
    pad = E_PAD - E
    srcp = jnp.concatenate([src, jnp.zeros((pad,), jnp.int32)])
    dstp = jnp.concatenate([dst, jnp.full((pad,), DUMMY, jnp.int32)])

    ones_tab = jnp.ones((N, 128), jnp.float32)
    deg = _seg(ones_tab, srcp, dstp).reshape(NC, N_PAD, 128)
    a1 = _seg(x, srcp, dstp).reshape(NC, N_PAD, 128)
    h1 = _tc1(a1, deg, x, W1l, W1r, b1.reshape(1, H))
    a2 = _seg(h1, srcp, dstp).reshape(NC, N_PAD, 128)
    h = _tc2(a2, deg, h1, x, W2l, W2r, b2.reshape(1, H),
             Wres, bres.reshape(1, H))
    a3 = _seg(h, srcp, dstp).reshape(NC, N_PAD, 128)
    mu, logstd = _tc3(a3, deg, h, Wmul, Wmur, bmu.reshape(1, O),
                      Wlsl, Wlsr, bls.reshape(1, O))
    return (mu, logstd)


# speedup vs baseline: 10.8515x; 3.2517x over previous
import jax
import jax.numpy as jnp
from jax import lax
from jax.experimental import pallas as pl
from jax.experimental.pallas import tpu as pltpu
from jax.experimental.pallas import tpu_sc as plsc

N = 10000
E = 320000
D = 128
H = 128
O = 64

NC = 2
NS = 16
NW = NC * NS
K = 100
KB = 2
NBLK = 50
EPW = NBLK * KB * K
N_PAD = 10240
RPS = N_PAD // NS
ZB = 16

_mesh = plsc.VectorSubcoreMesh(core_axis_name="c", subcore_axis_name="s")


def _zero_acc(zf, acc, rbase):
    zero16 = jnp.zeros((16,), dtype=jnp.float32)
    for i in range(ZB):
        for j in range(8):
            zf[i, pl.ds(j * 16, 16)] = zero16

    def zloop(i, _):
        pltpu.sync_copy(zf, acc.at[pl.ds(rbase + i * ZB, ZB)])
        return 0
    lax.fori_loop(0, RPS // ZB, zloop, 0)


def _seg_body(h_hbm, eidx_hbm, out_hbm, ib0, ib1, rows0, rows1, zf, acc,
              is0, is1, gs0, gs1):
    cid = lax.axis_index("c")
    sid = lax.axis_index("s")
    wid = sid * NC + cid
    rbase = sid * RPS

    _zero_acc(zf, acc, rbase)
    plsc.subcore_barrier()

    ibs = (ib0, ib1)
    isems = (is0, is1)
    rbufs = (rows0, rows1)
    gsems = (gs0, gs1)
    base = wid * NBLK

    def fetch_block(b, p):
        pltpu.async_copy(eidx_hbm.at[base + b], ibs[p], isems[p])

    def fetch_wait(b, p):
        pltpu.make_async_copy(eidx_hbm.at[base + b], ibs[p], isems[p]).wait()

    def gstart(p, j, rp):
        pltpu.async_copy(h_hbm.at[ibs[p].at[j]], rbufs[rp], gsems[rp])

    def gwait(p, j, rp):
        pltpu.make_async_copy(h_hbm.at[ibs[p].at[j]], rbufs[rp],
                              gsems[rp]).wait()

    def scat(p, j, rp):
        pltpu.sync_copy(rbufs[rp], acc.at[ibs[p].at[KB + j]], add=True)

    fetch_block(0, 0)
    fetch_wait(0, 0)
    gstart(0, 0, 0)

    def do_block(b, p):
        @pl.when(b < NBLK - 1)
        def _():
            fetch_block(b + 1, p ^ 1)
        gstart(p, 1, 1)
        gwait(p, 0, 0)
        scat(p, 0, 0)

        @pl.when(b < NBLK - 1)
        def _():
            fetch_wait(b + 1, p ^ 1)
            gstart(p ^ 1, 0, 0)
        gwait(p, 1, 1)
        scat(p, 1, 1)

    def body(b2, _):
        do_block(b2 * 2, 0)
        do_block(b2 * 2 + 1, 1)
        return 0
    lax.fori_loop(0, NBLK // 2, body, 0)

    plsc.subcore_barrier()

    pltpu.sync_copy(acc.at[pl.ds(rbase, RPS)],
                    out_hbm.at[pl.ds(cid * N_PAD + rbase, RPS)])


_seg = pl.kernel(
    _seg_body,
    out_type=jax.ShapeDtypeStruct((NC * N_PAD, 128), jnp.float32),
    mesh=_mesh,
    scratch_types=[
        pltpu.VMEM((2 * KB, K), jnp.int32),
        pltpu.VMEM((2 * KB, K), jnp.int32),
        pltpu.VMEM((K, 128), jnp.float32),
        pltpu.VMEM((K, 128), jnp.float32),
        pltpu.VMEM((ZB, 128), jnp.float32),
        pltpu.VMEM_SHARED((N_PAD, 128), jnp.float32),
        pltpu.SemaphoreType.DMA,
        pltpu.SemaphoreType.DMA,
        pltpu.SemaphoreType.DMA,
        pltpu.SemaphoreType.DMA,
    ],
)


def _deg_body(eidx_hbm, out_hbm, ib0, ib1, ones_v, zf, acc, is0, is1):
    cid = lax.axis_index("c")
    sid = lax.axis_index("s")
    wid = sid * NC + cid
    rbase = sid * RPS

    _zero_acc(zf, acc, rbase)
    one16 = jnp.full((16,), 1.0, dtype=jnp.float32)
    for i in range(K):
        for j in range(8):
            ones_v[i, pl.ds(j * 16, 16)] = one16
    plsc.subcore_barrier()

    ibs = (ib0, ib1)
    isems = (is0, is1)
    base = wid * NBLK

    def fetch_block(b, p):
        pltpu.async_copy(eidx_hbm.at[base + b], ibs[p], isems[p])

    def fetch_wait(b, p):
        pltpu.make_async_copy(eidx_hbm.at[base + b], ibs[p], isems[p]).wait()

    fetch_block(0, 0)

    def do_block(b, p):
        @pl.when(b < NBLK - 1)
        def _():
            fetch_block(b + 1, p ^ 1)
        fetch_wait(b, p)
        pltpu.sync_copy(ones_v, acc.at[ibs[p].at[KB]], add=True)
        pltpu.sync_copy(ones_v, acc.at[ibs[p].at[KB + 1]], add=True)

    def body(b2, _):
        do_block(b2 * 2, 0)
        do_block(b2 * 2 + 1, 1)
        return 0
    lax.fori_loop(0, NBLK // 2, body, 0)

    plsc.subcore_barrier()
    pltpu.sync_copy(acc.at[pl.ds(rbase, RPS)],
                    out_hbm.at[pl.ds(cid * N_PAD + rbase, RPS)])


_deg_kernel = pl.kernel(
    _deg_body,
    out_type=jax.ShapeDtypeStruct((NC * N_PAD, 128), jnp.float32),
    mesh=_mesh,
    scratch_types=[
        pltpu.VMEM((2 * KB, K), jnp.int32),
        pltpu.VMEM((2 * KB, K), jnp.int32),
        pltpu.VMEM((K, 128), jnp.float32),
        pltpu.VMEM((ZB, 128), jnp.float32),
        pltpu.VMEM_SHARED((N_PAD, 128), jnp.float32),
        pltpu.SemaphoreType.DMA,
        pltpu.SemaphoreType.DMA,
    ],
)


BN = 1000
GRID = N // BN


def _mean(a_ref, deg_ref):
    s = a_ref[0] + a_ref[1]
    deg = deg_ref[0, :, 0:1] + deg_ref[1, :, 0:1]
    return s * (1.0 / jnp.maximum(deg, 1.0))


def _tc1_body(a_ref, deg_ref, x_ref, wl_ref, wr_ref, b_ref, o_ref):
    mean = _mean(a_ref, deg_ref)
    h = (jnp.dot(mean, wl_ref[...], preferred_element_type=jnp.float32)
         + jnp.dot(x_ref[...], wr_ref[...], preferred_element_type=jnp.float32)
         + b_ref[...])
    o_ref[...] = jnp.maximum(h, 0.0)


def _tc2_body(a_ref, deg_ref, h1_ref, x_ref, wl_ref, wr_ref, b_ref,
              wres_ref, bres_ref, o_ref):
    mean = _mean(a_ref, deg_ref)
    h = (jnp.dot(mean, wl_ref[...], preferred_element_type=jnp.float32)
         + jnp.dot(h1_ref[...], wr_ref[...], preferred_element_type=jnp.float32)
         + b_ref[...])
    o_ref[...] = (jnp.maximum(h, 0.0)
                  + jnp.dot(x_ref[...], wres_ref[...],
                            preferred_element_type=jnp.float32)
                  + bres_ref[...])


def _tc3_body(a_ref, deg_ref, h_ref, wmul_ref, wmur_ref, bmu_ref,
              wlsl_ref, wlsr_ref, bls_ref, mu_ref, ls_ref):
    mean = _mean(a_ref, deg_ref)
    mu_ref[...] = (jnp.dot(mean, wmul_ref[...], preferred_element_type=jnp.float32)
                   + jnp.dot(h_ref[...], wmur_ref[...],
                             preferred_element_type=jnp.float32)
                   + bmu_ref[...])
    ls_ref[...] = (jnp.dot(mean, wlsl_ref[...], preferred_element_type=jnp.float32)
                   + jnp.dot(h_ref[...], wlsr_ref[...],
                             preferred_element_type=jnp.float32)
                   + bls_ref[...])


def _a_spec():
    return pl.BlockSpec((NC, BN, 128), lambda i: (0, i, 0))


def _deg_spec():
    return pl.BlockSpec((NC, BN, 128), lambda i: (0, i, 0))


def _row_spec(width):
    return pl.BlockSpec((BN, width), lambda i: (i, 0))


def _w_spec(r, c):
    return pl.BlockSpec((r, c), lambda i: (0, 0))


def _tc1(a, deg, x, wl, wr, b):
    return pl.pallas_call(
        _tc1_body,
        out_shape=jax.ShapeDtypeStruct((N, H), jnp.float32),
        grid=(GRID,),
        in_specs=[_a_spec(), _deg_spec(), _row_spec(D),
                  _w_spec(D, H), _w_spec(D, H), _w_spec(1, H)],
        out_specs=_row_spec(H),
    )(a, deg, x, wl, wr, b)


def _tc2(a, deg, h1, x, wl, wr, b, wres, bres):
    return pl.pallas_call(
        _tc2_body,
        out_shape=jax.ShapeDtypeStruct((N, H), jnp.float32),
        grid=(GRID,),
        in_specs=[_a_spec(), _deg_spec(), _row_spec(H), _row_spec(D),
                  _w_spec(H, H), _w_spec(H, H), _w_spec(1, H),
                  _w_spec(D, H), _w_spec(1, H)],
        out_specs=_row_spec(H),
    )(a, deg, h1, x, wl, wr, b, wres, bres)


def _tc3(a, deg, h, wmul, wmur, bmu, wlsl, wlsr, bls):
    return pl.pallas_call(
        _tc3_body,
        out_shape=(jax.ShapeDtypeStruct((N, O), jnp.float32),
                   jax.ShapeDtypeStruct((N, O), jnp.float32)),
        grid=(GRID,),
        in_specs=[_a_spec(), _deg_spec(), _row_spec(H),
                  _w_spec(H, O), _w_spec(H, O), _w_spec(1, O),
                  _w_spec(H, O), _w_spec(H, O), _w_spec(1, O)],
        out_specs=(_row_spec(O), _row_spec(O)),
    )(a, deg, h, wmul, wmur, bmu, wlsl, wlsr, bls)


def kernel(x, edge_index, W1l, W1r, b1, W2l, W2r, b2, Wres, bres,
           Wmul, Wmur, bmu, Wlsl, Wlsr, bls):
    src = edge_index[0].astype(jnp.int32)
    dst = edge_index[1].astype(jnp.int32)
    src4 = src.reshape(NW, NBLK, KB, K)
    dst4 = dst.reshape(NW, NBLK, KB, K)
    eidx = jnp.concatenate([src4, dst4], axis=2).reshape(NW * NBLK, 2 * KB, K)

    deg = _deg_kernel(eidx).reshape(NC, N_PAD, 128)
    a1 = _seg(x, eidx).reshape(NC, N_PAD, 128)
    h1 = _tc1(a1, deg, x, W1l, W1r, b1.reshape(1, H))
    a2 = _seg(h1, eidx).reshape(NC, N_PAD, 128)
    h = _tc2(a2, deg, h1, x, W2l, W2r, b2.reshape(1, H),
             Wres, bres.reshape(1, H))
    a3 = _seg(h, eidx).reshape(NC, N_PAD, 128)
    mu, logstd = _tc3(a3, deg, h, Wmul, Wmur, bmu.reshape(1, O),
                      Wlsl, Wlsr, bls.reshape(1, O))
    return (mu, logstd)


# speedup vs baseline: 11.1619x; 1.0286x over previous
import jax
import jax.numpy as jnp
from jax import lax
from jax.experimental import pallas as pl
from jax.experimental.pallas import tpu as pltpu
from jax.experimental.pallas import tpu_sc as plsc

N = 10000
E = 320000
D = 128
H = 128
O = 64

NC = 2
NS = 16
NW = NC * NS
K = 100
KB = 2
NBLK = 50
EPW = NBLK * KB * K
N_PAD = 10240
RPS = N_PAD // NS
ZB = 16

_mesh = plsc.VectorSubcoreMesh(core_axis_name="c", subcore_axis_name="s")


def _zero_acc(zf, acc, rbase, zsem):
    zero16 = jnp.zeros((16,), dtype=jnp.float32)
    for i in range(ZB):
        for j in range(8):
            zf[i, pl.ds(j * 16, 16)] = zero16

    def zloop(i, _):
        pltpu.async_copy(zf, acc.at[pl.ds(rbase + i * ZB, ZB)], zsem)
        return 0
    lax.fori_loop(0, RPS // ZB, zloop, 0)

    def zdrain(i, _):
        pltpu.make_async_copy(zf, acc.at[pl.ds(rbase + i * ZB, ZB)],
                              zsem).wait()
        return 0
    lax.fori_loop(0, RPS // ZB, zdrain, 0)


def _seg_body(h_hbm, eidx_hbm, out_hbm, ib0, ib1, rows0, rows1, zf, acc,
              is0, is1, gs0, gs1, zsem):
    cid = lax.axis_index("c")
    sid = lax.axis_index("s")
    wid = sid * NC + cid
    rbase = sid * RPS

    ibs = (ib0, ib1)
    isems = (is0, is1)
    rbufs = (rows0, rows1)
    gsems = (gs0, gs1)
    base = wid * NBLK

    def fetch_block(b, p):
        pltpu.async_copy(eidx_hbm.at[base + b], ibs[p], isems[p])

    def fetch_wait(b, p):
        pltpu.make_async_copy(eidx_hbm.at[base + b], ibs[p], isems[p]).wait()

    def gstart(p, j, rp):
        pltpu.async_copy(h_hbm.at[ibs[p].at[j]], rbufs[rp], gsems[rp])

    def gwait(p, j, rp):
        pltpu.make_async_copy(h_hbm.at[ibs[p].at[j]], rbufs[rp],
                              gsems[rp]).wait()

    def scat(p, j, rp):
        pltpu.sync_copy(rbufs[rp], acc.at[ibs[p].at[KB + j]], add=True)

    fetch_block(0, 0)
    fetch_wait(0, 0)
    gstart(0, 0, 0)
    _zero_acc(zf, acc, rbase, zsem)
    plsc.subcore_barrier()

    def do_block(b, p):
        @pl.when(b < NBLK - 1)
        def _():
            fetch_block(b + 1, p ^ 1)
        gstart(p, 1, 1)
        gwait(p, 0, 0)
        scat(p, 0, 0)

        @pl.when(b < NBLK - 1)
        def _():
            fetch_wait(b + 1, p ^ 1)
            gstart(p ^ 1, 0, 0)
        gwait(p, 1, 1)
        scat(p, 1, 1)

    def body(b2, _):
        do_block(b2 * 2, 0)
        do_block(b2 * 2 + 1, 1)
        return 0
    lax.fori_loop(0, NBLK // 2, body, 0)

    plsc.subcore_barrier()

    pltpu.sync_copy(acc.at[pl.ds(rbase, RPS)],
                    out_hbm.at[pl.ds(cid * N_PAD + rbase, RPS)])


_seg = pl.kernel(
    _seg_body,
    out_type=jax.ShapeDtypeStruct((NC * N_PAD, 128), jnp.float32),
    mesh=_mesh,
    scratch_types=[
        pltpu.VMEM((2 * KB, K), jnp.int32),
        pltpu.VMEM((2 * KB, K), jnp.int32),
        pltpu.VMEM((K, 128), jnp.float32),
        pltpu.VMEM((K, 128), jnp.float32),
        pltpu.VMEM((ZB, 128), jnp.float32),
        pltpu.VMEM_SHARED((N_PAD, 128), jnp.float32),
        pltpu.SemaphoreType.DMA,
        pltpu.SemaphoreType.DMA,
        pltpu.SemaphoreType.DMA,
        pltpu.SemaphoreType.DMA,
        pltpu.SemaphoreType.DMA,
    ],
)


def _deg_body(eidx_hbm, out_hbm, ib0, ib1, ones_v, zf, acc, is0, is1, zsem):
    cid = lax.axis_index("c")
    sid = lax.axis_index("s")
    wid = sid * NC + cid
    rbase = sid * RPS

    ibs = (ib0, ib1)
    isems = (is0, is1)
    base = wid * NBLK

    def fetch_block(b, p):
        pltpu.async_copy(eidx_hbm.at[base + b], ibs[p], isems[p])

    def fetch_wait(b, p):
        pltpu.make_async_copy(eidx_hbm.at[base + b], ibs[p], isems[p]).wait()

    fetch_block(0, 0)
    _zero_acc(zf, acc, rbase, zsem)
    one16 = jnp.full((16,), 1.0, dtype=jnp.float32)
    for i in range(K):
        for j in range(8):
            ones_v[i, pl.ds(j * 16, 16)] = one16
    plsc.subcore_barrier()

    def do_block(b, p):
        @pl.when(b < NBLK - 1)
        def _():
            fetch_block(b + 1, p ^ 1)
        fetch_wait(b, p)
        pltpu.sync_copy(ones_v, acc.at[ibs[p].at[KB]], add=True)
        pltpu.sync_copy(ones_v, acc.at[ibs[p].at[KB + 1]], add=True)

    def body(b2, _):
        do_block(b2 * 2, 0)
        do_block(b2 * 2 + 1, 1)
        return 0
    lax.fori_loop(0, NBLK // 2, body, 0)

    plsc.subcore_barrier()
    pltpu.sync_copy(acc.at[pl.ds(rbase, RPS)],
                    out_hbm.at[pl.ds(cid * N_PAD + rbase, RPS)])


_deg_kernel = pl.kernel(
    _deg_body,
    out_type=jax.ShapeDtypeStruct((NC * N_PAD, 128), jnp.float32),
    mesh=_mesh,
    scratch_types=[
        pltpu.VMEM((2 * KB, K), jnp.int32),
        pltpu.VMEM((2 * KB, K), jnp.int32),
        pltpu.VMEM((K, 128), jnp.float32),
        pltpu.VMEM((ZB, 128), jnp.float32),
        pltpu.VMEM_SHARED((N_PAD, 128), jnp.float32),
        pltpu.SemaphoreType.DMA,
        pltpu.SemaphoreType.DMA,
        pltpu.SemaphoreType.DMA,
    ],
)


BN = 1000
GRID = N // BN


def _mean(a_ref, deg_ref):
    s = a_ref[0] + a_ref[1]
    deg = deg_ref[0, :, 0:1] + deg_ref[1, :, 0:1]
    return s * (1.0 / jnp.maximum(deg, 1.0))


def _tc1_body(a_ref, deg_ref, x_ref, wl_ref, wr_ref, b_ref, o_ref):
    mean = _mean(a_ref, deg_ref)
    h = (jnp.dot(mean, wl_ref[...], preferred_element_type=jnp.float32)
         + jnp.dot(x_ref[...], wr_ref[...], preferred_element_type=jnp.float32)
         + b_ref[...])
    o_ref[...] = jnp.maximum(h, 0.0)


def _tc2_body(a_ref, deg_ref, h1_ref, x_ref, wl_ref, wr_ref, b_ref,
              wres_ref, bres_ref, o_ref):
    mean = _mean(a_ref, deg_ref)
    h = (jnp.dot(mean, wl_ref[...], preferred_element_type=jnp.float32)
         + jnp.dot(h1_ref[...], wr_ref[...], preferred_element_type=jnp.float32)
         + b_ref[...])
    o_ref[...] = (jnp.maximum(h, 0.0)
                  + jnp.dot(x_ref[...], wres_ref[...],
                            preferred_element_type=jnp.float32)
                  + bres_ref[...])


def _tc3_body(a_ref, deg_ref, h_ref, wmul_ref, wmur_ref, bmu_ref,
              wlsl_ref, wlsr_ref, bls_ref, mu_ref, ls_ref):
    mean = _mean(a_ref, deg_ref)
    mu_ref[...] = (jnp.dot(mean, wmul_ref[...], preferred_element_type=jnp.float32)
                   + jnp.dot(h_ref[...], wmur_ref[...],
                             preferred_element_type=jnp.float32)
                   + bmu_ref[...])
    ls_ref[...] = (jnp.dot(mean, wlsl_ref[...], preferred_element_type=jnp.float32)
                   + jnp.dot(h_ref[...], wlsr_ref[...],
                             preferred_element_type=jnp.float32)
                   + bls_ref[...])


def _a_spec():
    return pl.BlockSpec((NC, BN, 128), lambda i: (0, i, 0))


def _deg_spec():
    return pl.BlockSpec((NC, BN, 128), lambda i: (0, i, 0))


def _row_spec(width):
    return pl.BlockSpec((BN, width), lambda i: (i, 0))


def _w_spec(r, c):
    return pl.BlockSpec((r, c), lambda i: (0, 0))


def _tc1(a, deg, x, wl, wr, b):
    return pl.pallas_call(
        _tc1_body,
        out_shape=jax.ShapeDtypeStruct((N, H), jnp.float32),
        grid=(GRID,),
        in_specs=[_a_spec(), _deg_spec(), _row_spec(D),
                  _w_spec(D, H), _w_spec(D, H), _w_spec(1, H)],
        out_specs=_row_spec(H),
    )(a, deg, x, wl, wr, b)


def _tc2(a, deg, h1, x, wl, wr, b, wres, bres):
    return pl.pallas_call(
        _tc2_body,
        out_shape=jax.ShapeDtypeStruct((N, H), jnp.float32),
        grid=(GRID,),
        in_specs=[_a_spec(), _deg_spec(), _row_spec(H), _row_spec(D),
                  _w_spec(H, H), _w_spec(H, H), _w_spec(1, H),
                  _w_spec(D, H), _w_spec(1, H)],
        out_specs=_row_spec(H),
    )(a, deg, h1, x, wl, wr, b, wres, bres)


def _tc3(a, deg, h, wmul, wmur, bmu, wlsl, wlsr, bls):
    return pl.pallas_call(
        _tc3_body,
        out_shape=(jax.ShapeDtypeStruct((N, O), jnp.float32),
                   jax.ShapeDtypeStruct((N, O), jnp.float32)),
        grid=(GRID,),
        in_specs=[_a_spec(), _deg_spec(), _row_spec(H),
                  _w_spec(H, O), _w_spec(H, O), _w_spec(1, O),
                  _w_spec(H, O), _w_spec(H, O), _w_spec(1, O)],
        out_specs=(_row_spec(O), _row_spec(O)),
    )(a, deg, h, wmul, wmur, bmu, wlsl, wlsr, bls)


def kernel(x, edge_index, W1l, W1r, b1, W2l, W2r, b2, Wres, bres,
           Wmul, Wmur, bmu, Wlsl, Wlsr, bls):
    src = edge_index[0].astype(jnp.int32)
    dst = edge_index[1].astype(jnp.int32)
    src4 = src.reshape(NW, NBLK, KB, K)
    dst4 = dst.reshape(NW, NBLK, KB, K)
    eidx = jnp.concatenate([src4, dst4], axis=2).reshape(NW * NBLK, 2 * KB, K)

    deg = _deg_kernel(eidx).reshape(NC, N_PAD, 128)
    a1 = _seg(x, eidx).reshape(NC, N_PAD, 128)
    h1 = _tc1(a1, deg, x, W1l, W1r, b1.reshape(1, H))
    a2 = _seg(h1, eidx).reshape(NC, N_PAD, 128)
    h = _tc2(a2, deg, h1, x, W2l, W2r, b2.reshape(1, H),
             Wres, bres.reshape(1, H))
    a3 = _seg(h, eidx).reshape(NC, N_PAD, 128)
    mu, logstd = _tc3(a3, deg, h, Wmul, Wmur, bmu.reshape(1, O),
                      Wlsl, Wlsr, bls.reshape(1, O))
    return (mu, logstd)


# speedup vs baseline: 11.2518x; 1.0081x over previous
import jax
import jax.numpy as jnp
from jax import lax
from jax.experimental import pallas as pl
from jax.experimental.pallas import tpu as pltpu
from jax.experimental.pallas import tpu_sc as plsc

N = 10000
E = 320000
D = 128
H = 128
O = 64

NC = 2
NS = 16
NW = NC * NS
K = 100
KB = 2
NBLK = 50
EPW = NBLK * KB * K
N_PAD = 10240
RPS = N_PAD // NS
ZB = 16

_mesh = plsc.VectorSubcoreMesh(core_axis_name="c", subcore_axis_name="s")


def _zero_acc(zf, acc, rbase, zsem):
    zero16 = jnp.zeros((16,), dtype=jnp.float32)
    for i in range(ZB):
        for j in range(8):
            zf[i, pl.ds(j * 16, 16)] = zero16

    def zloop(i, _):
        pltpu.async_copy(zf, acc.at[pl.ds(rbase + i * ZB, ZB)], zsem)
        return 0
    lax.fori_loop(0, RPS // ZB, zloop, 0)

    def zdrain(i, _):
        pltpu.make_async_copy(zf, acc.at[pl.ds(rbase + i * ZB, ZB)],
                              zsem).wait()
        return 0
    lax.fori_loop(0, RPS // ZB, zdrain, 0)


def _seg_body(h_hbm, eidx_hbm, out_hbm, ib0, ib1, rows0, rows1, zf, acc,
              is0, is1, gs0, gs1, zsem):
    cid = lax.axis_index("c")
    sid = lax.axis_index("s")
    wid = sid * NC + cid
    rbase = sid * RPS

    ibs = (ib0, ib1)
    isems = (is0, is1)
    rbufs = (rows0, rows1)
    gsems = (gs0, gs1)
    base = wid * NBLK

    def fetch_block(b, p):
        pltpu.async_copy(eidx_hbm.at[base + b], ibs[p], isems[p])

    def fetch_wait(b, p):
        pltpu.make_async_copy(eidx_hbm.at[base + b], ibs[p], isems[p]).wait()

    def gstart(p, j, rp):
        pltpu.async_copy(h_hbm.at[ibs[p].at[j]], rbufs[rp], gsems[rp])

    def gwait(p, j, rp):
        pltpu.make_async_copy(h_hbm.at[ibs[p].at[j]], rbufs[rp],
                              gsems[rp]).wait()

    def scat(p, j, rp):
        pltpu.sync_copy(rbufs[rp], acc.at[ibs[p].at[KB + j]], add=True)

    fetch_block(0, 0)
    fetch_wait(0, 0)
    gstart(0, 0, 0)
    _zero_acc(zf, acc, rbase, zsem)
    plsc.subcore_barrier()

    def do_block(b, p):
        @pl.when(b < NBLK - 1)
        def _():
            fetch_block(b + 1, p ^ 1)
        gstart(p, 1, 1)
        gwait(p, 0, 0)
        scat(p, 0, 0)

        @pl.when(b < NBLK - 1)
        def _():
            fetch_wait(b + 1, p ^ 1)
            gstart(p ^ 1, 0, 0)
        gwait(p, 1, 1)
        scat(p, 1, 1)

    def body(b2, _):
        do_block(b2 * 2, 0)
        do_block(b2 * 2 + 1, 1)
        return 0
    lax.fori_loop(0, NBLK // 2, body, 0)

    plsc.subcore_barrier()

    pltpu.sync_copy(acc.at[pl.ds(rbase, RPS)],
                    out_hbm.at[pl.ds(cid * N_PAD + rbase, RPS)])


_seg = pl.kernel(
    _seg_body,
    out_type=jax.ShapeDtypeStruct((NC * N_PAD, 128), jnp.float32),
    mesh=_mesh,
    scratch_types=[
        pltpu.VMEM((2 * KB, K), jnp.int32),
        pltpu.VMEM((2 * KB, K), jnp.int32),
        pltpu.VMEM((K, 128), jnp.float32),
        pltpu.VMEM((K, 128), jnp.float32),
        pltpu.VMEM((ZB, 128), jnp.float32),
        pltpu.VMEM_SHARED((N_PAD, 128), jnp.float32),
        pltpu.SemaphoreType.DMA,
        pltpu.SemaphoreType.DMA,
        pltpu.SemaphoreType.DMA,
        pltpu.SemaphoreType.DMA,
        pltpu.SemaphoreType.DMA,
    ],
)


def _seg1_body(h_hbm, eidx_hbm, deg_hbm, out_hbm, ib0, ib1, rows0, rows1,
               zf, acc, is0, is1, gs0, gs1, zsem, wsem):
    cid = lax.axis_index("c")
    sid = lax.axis_index("s")
    wid = sid * NC + cid
    rbase = sid * RPS

    ibs = (ib0, ib1)
    isems = (is0, is1)
    rbufs = (rows0, rows1)
    gsems = (gs0, gs1)
    base = wid * NBLK

    def fetch_block(b, p):
        pltpu.async_copy(eidx_hbm.at[base + b], ibs[p], isems[p])

    def fetch_wait(b, p):
        pltpu.make_async_copy(eidx_hbm.at[base + b], ibs[p], isems[p]).wait()

    def gstart(p, j, rp):
        pltpu.async_copy(h_hbm.at[ibs[p].at[j]], rbufs[rp], gsems[rp])

    def gwait(p, j, rp):
        pltpu.make_async_copy(h_hbm.at[ibs[p].at[j]], rbufs[rp],
                              gsems[rp]).wait()

    def scat(p, j, rp):
        pltpu.sync_copy(rbufs[rp], acc.at[ibs[p].at[KB + j]], add=True)

    fetch_block(0, 0)
    _zero_acc(zf, acc, rbase, zsem)
    one16 = jnp.full((16,), 1.0, dtype=jnp.float32)
    for i in range(K):
        for j in range(8):
            rows0[i, pl.ds(j * 16, 16)] = one16
    plsc.subcore_barrier()

    def deg_block(b, p):
        @pl.when(b < NBLK - 1)
        def _():
            fetch_block(b + 1, p ^ 1)
        fetch_wait(b, p)
        pltpu.sync_copy(rows0, acc.at[ibs[p].at[KB]], add=True)
        pltpu.sync_copy(rows0, acc.at[ibs[p].at[KB + 1]], add=True)

    def deg_loop(b2, _):
        deg_block(b2 * 2, 0)
        deg_block(b2 * 2 + 1, 1)
        return 0
    lax.fori_loop(0, NBLK // 2, deg_loop, 0)

    plsc.subcore_barrier()
    wb = pltpu.async_copy(acc.at[pl.ds(rbase, RPS)],
                          deg_hbm.at[pl.ds(cid * N_PAD + rbase, RPS)], wsem)

    fetch_block(0, 0)
    fetch_wait(0, 0)
    gstart(0, 0, 0)
    wb.wait()
    _zero_acc(zf, acc, rbase, zsem)
    plsc.subcore_barrier()

    def do_block(b, p):
        @pl.when(b < NBLK - 1)
        def _():
            fetch_block(b + 1, p ^ 1)
        gstart(p, 1, 1)
        gwait(p, 0, 0)
        scat(p, 0, 0)

        @pl.when(b < NBLK - 1)
        def _():
            fetch_wait(b + 1, p ^ 1)
            gstart(p ^ 1, 0, 0)
        gwait(p, 1, 1)
        scat(p, 1, 1)

    def body(b2, _):
        do_block(b2 * 2, 0)
        do_block(b2 * 2 + 1, 1)
        return 0
    lax.fori_loop(0, NBLK // 2, body, 0)

    plsc.subcore_barrier()
    pltpu.sync_copy(acc.at[pl.ds(rbase, RPS)],
                    out_hbm.at[pl.ds(cid * N_PAD + rbase, RPS)])


_seg1 = pl.kernel(
    _seg1_body,
    out_type=(jax.ShapeDtypeStruct((NC * N_PAD, 128), jnp.float32),
              jax.ShapeDtypeStruct((NC * N_PAD, 128), jnp.float32)),
    mesh=_mesh,
    scratch_types=[
        pltpu.VMEM((2 * KB, K), jnp.int32),
        pltpu.VMEM((2 * KB, K), jnp.int32),
        pltpu.VMEM((K, 128), jnp.float32),
        pltpu.VMEM((K, 128), jnp.float32),
        pltpu.VMEM((ZB, 128), jnp.float32),
        pltpu.VMEM_SHARED((N_PAD, 128), jnp.float32),
        pltpu.SemaphoreType.DMA,
        pltpu.SemaphoreType.DMA,
        pltpu.SemaphoreType.DMA,
        pltpu.SemaphoreType.DMA,
        pltpu.SemaphoreType.DMA,
        pltpu.SemaphoreType.DMA,
    ],
)


BN = 1000
GRID = N // BN


def _mean(a_ref, deg_ref):
    s = a_ref[0] + a_ref[1]
    deg = deg_ref[0, :, 0:1] + deg_ref[1, :, 0:1]
    return s * (1.0 / jnp.maximum(deg, 1.0))


def _tc1_body(a_ref, deg_ref, x_ref, wl_ref, wr_ref, b_ref, o_ref):
    mean = _mean(a_ref, deg_ref)
    h = (jnp.dot(mean, wl_ref[...], preferred_element_type=jnp.float32)
         + jnp.dot(x_ref[...], wr_ref[...], preferred_element_type=jnp.float32)
         + b_ref[...])
    o_ref[...] = jnp.maximum(h, 0.0)


def _tc2_body(a_ref, deg_ref, h1_ref, x_ref, wl_ref, wr_ref, b_ref,
              wres_ref, bres_ref, o_ref):
    mean = _mean(a_ref, deg_ref)
    h = (jnp.dot(mean, wl_ref[...], preferred_element_type=jnp.float32)
         + jnp.dot(h1_ref[...], wr_ref[...], preferred_element_type=jnp.float32)
         + b_ref[...])
    o_ref[...] = (jnp.maximum(h, 0.0)
                  + jnp.dot(x_ref[...], wres_ref[...],
                            preferred_element_type=jnp.float32)
                  + bres_ref[...])


def _tc3_body(a_ref, deg_ref, h_ref, wmul_ref, wmur_ref, bmu_ref,
              wlsl_ref, wlsr_ref, bls_ref, mu_ref, ls_ref):
    mean = _mean(a_ref, deg_ref)
    mu_ref[...] = (jnp.dot(mean, wmul_ref[...], preferred_element_type=jnp.float32)
                   + jnp.dot(h_ref[...], wmur_ref[...],
                             preferred_element_type=jnp.float32)
                   + bmu_ref[...])
    ls_ref[...] = (jnp.dot(mean, wlsl_ref[...], preferred_element_type=jnp.float32)
                   + jnp.dot(h_ref[...], wlsr_ref[...],
                             preferred_element_type=jnp.float32)
                   + bls_ref[...])


def _a_spec():
    return pl.BlockSpec((NC, BN, 128), lambda i: (0, i, 0))


def _deg_spec():
    return pl.BlockSpec((NC, BN, 128), lambda i: (0, i, 0))


def _row_spec(width):
    return pl.BlockSpec((BN, width), lambda i: (i, 0))


def _w_spec(r, c):
    return pl.BlockSpec((r, c), lambda i: (0, 0))


def _tc1(a, deg, x, wl, wr, b):
    return pl.pallas_call(
        _tc1_body,
        out_shape=jax.ShapeDtypeStruct((N, H), jnp.float32),
        grid=(GRID,),
        in_specs=[_a_spec(), _deg_spec(), _row_spec(D),
                  _w_spec(D, H), _w_spec(D, H), _w_spec(1, H)],
        out_specs=_row_spec(H),
    )(a, deg, x, wl, wr, b)


def _tc2(a, deg, h1, x, wl, wr, b, wres, bres):
    return pl.pallas_call(
        _tc2_body,
        out_shape=jax.ShapeDtypeStruct((N, H), jnp.float32),
        grid=(GRID,),
        in_specs=[_a_spec(), _deg_spec(), _row_spec(H), _row_spec(D),
                  _w_spec(H, H), _w_spec(H, H), _w_spec(1, H),
                  _w_spec(D, H), _w_spec(1, H)],
        out_specs=_row_spec(H),
    )(a, deg, h1, x, wl, wr, b, wres, bres)


def _tc3(a, deg, h, wmul, wmur, bmu, wlsl, wlsr, bls):
    return pl.pallas_call(
        _tc3_body,
        out_shape=(jax.ShapeDtypeStruct((N, O), jnp.float32),
                   jax.ShapeDtypeStruct((N, O), jnp.float32)),
        grid=(GRID,),
        in_specs=[_a_spec(), _deg_spec(), _row_spec(H),
                  _w_spec(H, O), _w_spec(H, O), _w_spec(1, O),
                  _w_spec(H, O), _w_spec(H, O), _w_spec(1, O)],
        out_specs=(_row_spec(O), _row_spec(O)),
    )(a, deg, h, wmul, wmur, bmu, wlsl, wlsr, bls)


def kernel(x, edge_index, W1l, W1r, b1, W2l, W2r, b2, Wres, bres,
           Wmul, Wmur, bmu, Wlsl, Wlsr, bls):
    src = edge_index[0].astype(jnp.int32)
    dst = edge_index[1].astype(jnp.int32)
    src4 = src.reshape(NW, NBLK, KB, K)
    dst4 = dst.reshape(NW, NBLK, KB, K)
    eidx = jnp.concatenate([src4, dst4], axis=2).reshape(NW * NBLK, 2 * KB, K)

    degf, a1f = _seg1(x, eidx)
    deg = degf.reshape(NC, N_PAD, 128)
    a1 = a1f.reshape(NC, N_PAD, 128)
    h1 = _tc1(a1, deg, x, W1l, W1r, b1.reshape(1, H))
    a2 = _seg(h1, eidx).reshape(NC, N_PAD, 128)
    h = _tc2(a2, deg, h1, x, W2l, W2r, b2.reshape(1, H),
             Wres, bres.reshape(1, H))
    a3 = _seg(h, eidx).reshape(NC, N_PAD, 128)
    mu, logstd = _tc3(a3, deg, h, Wmul, Wmur, bmu.reshape(1, O),
                      Wlsl, Wlsr, bls.reshape(1, O))
    return (mu, logstd)
